# Initial kernel scaffold; baseline (speedup 1.0000x reference)
import jax
import jax.numpy as jnp
from jax.experimental import pallas as pl


def kernel(atom, bond, bond_idx, W, b):
    raise NotImplementedError("write your pallas kernel here")


# speedup vs baseline: 1.5198x; 1.5198x over previous
import functools

import jax
import jax.numpy as jnp
from jax import lax
from jax.experimental import pallas as pl
from jax.experimental.pallas import tpu as pltpu
from jax.experimental.pallas import tpu_sc as plsc

N = 10000
M = 32
AD = 128
BD = 16
C = 2 * AD
NM = N * M
EPS = 1e-5

A_TILE = 40
R_TILE = A_TILE * M
GRID_ROWS = N // A_TILE

SC_CORES = 2
SC_SUBCORES = 16
NW = SC_CORES * SC_SUBCORES
ROWS_PER_W = NM // NW
CHUNK = 80
CHUNKS = ROWS_PER_W // CHUNK


def _softplus(x):
    return jnp.maximum(x, 0.0) + jnp.log1p(jnp.exp(-jnp.abs(x)))


def _proj_body(atom_ref, w1t_ref, w2t_ref, b_ref, p1_ref, p2_ref):
    x = atom_ref[...]
    p1_ref[...] = (
        jnp.dot(x, w1t_ref[...], preferred_element_type=jnp.float32) + b_ref[...]
    )
    p2_ref[...] = jnp.dot(x, w2t_ref[...], preferred_element_type=jnp.float32)


def _project(atom, w1t, w2t, b2):
    rows = 1000
    return pl.pallas_call(
        _proj_body,
        grid=(N // rows,),
        in_specs=[
            pl.BlockSpec((rows, AD), lambda i: (i, 0)),
            pl.BlockSpec((AD, C), lambda i: (0, 0)),
            pl.BlockSpec((AD, C), lambda i: (0, 0)),
            pl.BlockSpec((1, C), lambda i: (0, 0)),
        ],
        out_specs=[
            pl.BlockSpec((rows, C), lambda i: (i, 0)),
            pl.BlockSpec((rows, C), lambda i: (i, 0)),
        ],
        out_shape=[
            jax.ShapeDtypeStruct((N, C), jnp.float32),
            jax.ShapeDtypeStruct((N, C), jnp.float32),
        ],
    )(atom, w1t, w2t, b2)


def _gather_body(idx_hbm, p2_hbm, out_hbm, idx_v, rows_v, sem):
    wid = lax.axis_index("s") * SC_CORES + lax.axis_index("c")
    base = wid * ROWS_PER_W

    def step(c, carry):
        r0 = base + c * CHUNK
        pltpu.sync_copy(idx_hbm.at[pl.ds(r0, CHUNK)], idx_v)
        pltpu.async_copy(p2_hbm.at[idx_v], rows_v, sem).wait()
        pltpu.sync_copy(rows_v, out_hbm.at[pl.ds(r0, CHUNK)])
        return carry

    lax.fori_loop(0, CHUNKS, step, 0)


def _sc_gather(flat_idx, p2):
    mesh = plsc.VectorSubcoreMesh(core_axis_name="c", subcore_axis_name="s")
    f = pl.kernel(
        _gather_body,
        out_type=jax.ShapeDtypeStruct((NM, C), jnp.float32),
        mesh=mesh,
        scratch_types=[
            pltpu.VMEM((CHUNK,), jnp.int32),
            pltpu.VMEM((CHUNK, C), jnp.float32),
            pltpu.SemaphoreType.DMA,
        ],
    )
    return f(flat_idx, p2)


def _pre_activation(g2_ref, p1_ref, bond_ref, w3t_ref):
    p1 = p1_ref[...]
    p1rep = jnp.broadcast_to(p1[:, None, :], (A_TILE, M, C)).reshape(R_TILE, C)
    bw = jnp.dot(bond_ref[...], w3t_ref[...], preferred_element_type=jnp.float32)
    return g2_ref[...] + p1rep + bw


def _stats_body(g2_ref, p1_ref, bond_ref, w3t_ref, sums_ref):
    g = _pre_activation(g2_ref, p1_ref, bond_ref, w3t_ref)

    @pl.when(pl.program_id(0) == 0)
    def _():
        sums_ref[...] = jnp.zeros((8, C), jnp.float32)

    sums_ref[0:1, :] += jnp.sum(g, axis=0, keepdims=True)
    sums_ref[1:2, :] += jnp.sum(g * g, axis=0, keepdims=True)


def _stats(g2, p1, bond2, w3t):
    return pl.pallas_call(
        _stats_body,
        grid=(GRID_ROWS,),
        in_specs=[
            pl.BlockSpec((R_TILE, C), lambda i: (i, 0)),
            pl.BlockSpec((A_TILE, C), lambda i: (i, 0)),
            pl.BlockSpec((R_TILE, BD), lambda i: (i, 0)),
            pl.BlockSpec((BD, C), lambda i: (0, 0)),
        ],
        out_specs=pl.BlockSpec((8, C), lambda i: (0, 0)),
        out_shape=jax.ShapeDtypeStruct((8, C), jnp.float32),
    )(g2, p1, bond2, w3t)


def _apply_body(g2_ref, p1_ref, bond_ref, w3t_ref, sums_ref, s_ref, sums2_ref):
    g = _pre_activation(g2_ref, p1_ref, bond_ref, w3t_ref)
    mean = sums_ref[0:1, :] * (1.0 / NM)
    var = sums_ref[1:2, :] * (1.0 / NM) - mean * mean
    rstd = lax.rsqrt(var + EPS)
    gn = (g - mean) * rstd
    filt = jax.nn.sigmoid(gn[:, :AD])
    core = _softplus(gn[:, AD:])
    prod = filt * core
    s = jnp.sum(prod.reshape(A_TILE, M, AD), axis=1)
    s_ref[...] = s

    @pl.when(pl.program_id(0) == 0)
    def _():
        sums2_ref[...] = jnp.zeros((8, AD), jnp.float32)

    sums2_ref[0:1, :] += jnp.sum(s, axis=0, keepdims=True)
    sums2_ref[1:2, :] += jnp.sum(s * s, axis=0, keepdims=True)


def _apply(g2, p1, bond2, w3t, sums):
    return pl.pallas_call(
        _apply_body,
        grid=(GRID_ROWS,),
        in_specs=[
            pl.BlockSpec((R_TILE, C), lambda i: (i, 0)),
            pl.BlockSpec((A_TILE, C), lambda i: (i, 0)),
            pl.BlockSpec((R_TILE, BD), lambda i: (i, 0)),
            pl.BlockSpec((BD, C), lambda i: (0, 0)),
            pl.BlockSpec((8, C), lambda i: (0, 0)),
        ],
        out_specs=[
            pl.BlockSpec((A_TILE, AD), lambda i: (i, 0)),
            pl.BlockSpec((8, AD), lambda i: (0, 0)),
        ],
        out_shape=[
            jax.ShapeDtypeStruct((N, AD), jnp.float32),
            jax.ShapeDtypeStruct((8, AD), jnp.float32),
        ],
    )(g2, p1, bond2, w3t, sums)


def _final_body(atom_ref, s_ref, sums2_ref, out_ref):
    mean = sums2_ref[0:1, :] * (1.0 / N)
    var = sums2_ref[1:2, :] * (1.0 / N) - mean * mean
    rstd = lax.rsqrt(var + EPS)
    sn = (s_ref[...] - mean) * rstd
    out_ref[...] = _softplus(atom_ref[...] + sn)


def _final(atom, s, sums2):
    rows = 1000
    return pl.pallas_call(
        _final_body,
        grid=(N // rows,),
        in_specs=[
            pl.BlockSpec((rows, AD), lambda i: (i, 0)),
            pl.BlockSpec((rows, AD), lambda i: (i, 0)),
            pl.BlockSpec((8, AD), lambda i: (0, 0)),
        ],
        out_specs=pl.BlockSpec((rows, AD), lambda i: (i, 0)),
        out_shape=jax.ShapeDtypeStruct((N, AD), jnp.float32),
    )(atom, s, sums2)


def kernel(atom, bond, bond_idx, W, b):
    w1t = W[:, :AD].T.astype(jnp.float32)
    w2t = W[:, AD:2 * AD].T.astype(jnp.float32)
    w3t = W[:, 2 * AD:].T.astype(jnp.float32)
    b2 = b.reshape(1, C)

    p1, p2 = _project(atom, w1t, w2t, b2)
    g2 = _sc_gather(bond_idx.reshape(NM), p2)
    bond2 = bond.reshape(NM, BD)
    sums = _stats(g2, p1, bond2, w3t)
    s, sums2 = _apply(g2, p1, bond2, w3t, sums)
    return _final(atom, s, sums2)


# speedup vs baseline: 1.8197x; 1.1974x over previous
import functools

import jax
import jax.numpy as jnp
from jax import lax
from jax.experimental import pallas as pl
from jax.experimental.pallas import tpu as pltpu
from jax.experimental.pallas import tpu_sc as plsc

N = 10000
M = 32
AD = 128
BD = 16
C = 2 * AD
NM = N * M
EPS = 1e-5

A_TILE = 40
R_TILE = A_TILE * M
GRID_ROWS = N // A_TILE

SC_CORES = 2
SC_SUBCORES = 16
NW = SC_CORES * SC_SUBCORES
ROWS_PER_W = NM // NW
CHUNK = 80
CHUNKS = ROWS_PER_W // CHUNK


def _softplus(x):
    return jnp.maximum(x, 0.0) + jnp.log1p(jnp.exp(-jnp.abs(x)))


def _pack_bf16_pair(lo, hi):
    ul = lax.bitcast_convert_type(lo, jnp.uint32)
    uh = lax.bitcast_convert_type(hi, jnp.uint32)
    bl = (ul + jnp.uint32(0x7FFF) + ((ul >> 16) & jnp.uint32(1))) >> 16
    bh = (uh + jnp.uint32(0x7FFF) + ((uh >> 16) & jnp.uint32(1))) >> 16
    return lax.bitcast_convert_type(bl | (bh << 16), jnp.int32)


def _unpack_bf16_pair(w):
    u = lax.bitcast_convert_type(w, jnp.uint32)
    lo = lax.bitcast_convert_type(u << 16, jnp.float32)
    hi = lax.bitcast_convert_type(u & jnp.uint32(0xFFFF0000), jnp.float32)
    return lo, hi


def _proj_body(atom_ref, w1t_ref, w2t_ref, b_ref, p1_ref, p2_ref):
    x = atom_ref[...]
    p1_ref[...] = (
        jnp.dot(x, w1t_ref[...], preferred_element_type=jnp.float32) + b_ref[...]
    )
    p2 = jnp.dot(x, w2t_ref[...], preferred_element_type=jnp.float32)
    p2_ref[...] = _pack_bf16_pair(p2[:, :AD], p2[:, AD:])


def _project(atom, w1t, w2t, b2):
    rows = 1000
    return pl.pallas_call(
        _proj_body,
        grid=(N // rows,),
        in_specs=[
            pl.BlockSpec((rows, AD), lambda i: (i, 0)),
            pl.BlockSpec((AD, C), lambda i: (0, 0)),
            pl.BlockSpec((AD, C), lambda i: (0, 0)),
            pl.BlockSpec((1, C), lambda i: (0, 0)),
        ],
        out_specs=[
            pl.BlockSpec((rows, C), lambda i: (i, 0)),
            pl.BlockSpec((rows, AD), lambda i: (i, 0)),
        ],
        out_shape=[
            jax.ShapeDtypeStruct((N, C), jnp.float32),
            jax.ShapeDtypeStruct((N, AD), jnp.int32),
        ],
    )(atom, w1t, w2t, b2)


def _gather_body(idx_hbm, p2_hbm, out_hbm, idx_v, rows_v, sem):
    wid = lax.axis_index("s") * SC_CORES + lax.axis_index("c")
    base = wid * ROWS_PER_W

    def step(c, carry):
        r0 = base + c * CHUNK
        pltpu.sync_copy(idx_hbm.at[pl.ds(r0, CHUNK)], idx_v)
        pltpu.async_copy(p2_hbm.at[idx_v], rows_v, sem).wait()
        pltpu.sync_copy(rows_v, out_hbm.at[pl.ds(r0, CHUNK)])
        return carry

    lax.fori_loop(0, CHUNKS, step, 0)


def _sc_gather(flat_idx, p2):
    mesh = plsc.VectorSubcoreMesh(core_axis_name="c", subcore_axis_name="s")
    f = pl.kernel(
        _gather_body,
        out_type=jax.ShapeDtypeStruct((NM, AD), jnp.int32),
        mesh=mesh,
        scratch_types=[
            pltpu.VMEM((CHUNK,), jnp.int32),
            pltpu.VMEM((CHUNK, AD), jnp.int32),
            pltpu.SemaphoreType.DMA,
        ],
    )
    return f(flat_idx, p2)


def _pre_activation(g2_ref, p1_ref, bond_ref, w3t_ref):
    p1 = p1_ref[...]
    p1rep = jnp.broadcast_to(p1[:, None, :], (A_TILE, M, C)).reshape(R_TILE, C)
    bw = jnp.dot(
        bond_ref[...].reshape(R_TILE, BD),
        w3t_ref[...],
        preferred_element_type=jnp.float32,
    )
    base = p1rep + bw
    lo, hi = _unpack_bf16_pair(g2_ref[...])
    return lo + base[:, :AD], hi + base[:, AD:]


def _stats_body(g2_ref, p1_ref, bond_ref, w3t_ref, sums_ref):
    gl, gh = _pre_activation(g2_ref, p1_ref, bond_ref, w3t_ref)

    @pl.when(pl.program_id(0) == 0)
    def _():
        sums_ref[...] = jnp.zeros((8, C), jnp.float32)

    sums_ref[0:1, :AD] += jnp.sum(gl, axis=0, keepdims=True)
    sums_ref[0:1, AD:] += jnp.sum(gh, axis=0, keepdims=True)
    sums_ref[1:2, :AD] += jnp.sum(gl * gl, axis=0, keepdims=True)
    sums_ref[1:2, AD:] += jnp.sum(gh * gh, axis=0, keepdims=True)


def _stats(g2, p1, bond, w3t):
    return pl.pallas_call(
        _stats_body,
        grid=(GRID_ROWS,),
        in_specs=[
            pl.BlockSpec((R_TILE, AD), lambda i: (i, 0)),
            pl.BlockSpec((A_TILE, C), lambda i: (i, 0)),
            pl.BlockSpec((A_TILE, M, BD), lambda i: (i, 0, 0)),
            pl.BlockSpec((BD, C), lambda i: (0, 0)),
        ],
        out_specs=pl.BlockSpec((8, C), lambda i: (0, 0)),
        out_shape=jax.ShapeDtypeStruct((8, C), jnp.float32),
    )(g2, p1, bond, w3t)


def _apply_body(g2_ref, p1_ref, bond_ref, w3t_ref, sums_ref, s_ref, sums2_ref):
    gl, gh = _pre_activation(g2_ref, p1_ref, bond_ref, w3t_ref)
    mean = sums_ref[0:1, :] * (1.0 / NM)
    var = sums_ref[1:2, :] * (1.0 / NM) - mean * mean
    rstd = lax.rsqrt(var + EPS)
    filt = jax.nn.sigmoid((gl - mean[:, :AD]) * rstd[:, :AD])
    core = _softplus((gh - mean[:, AD:]) * rstd[:, AD:])
    prod = filt * core
    s = jnp.sum(prod.reshape(A_TILE, M, AD), axis=1)
    s_ref[...] = s

    @pl.when(pl.program_id(0) == 0)
    def _():
        sums2_ref[...] = jnp.zeros((8, AD), jnp.float32)

    sums2_ref[0:1, :] += jnp.sum(s, axis=0, keepdims=True)
    sums2_ref[1:2, :] += jnp.sum(s * s, axis=0, keepdims=True)


def _apply(g2, p1, bond, w3t, sums):
    return pl.pallas_call(
        _apply_body,
        grid=(GRID_ROWS,),
        in_specs=[
            pl.BlockSpec((R_TILE, AD), lambda i: (i, 0)),
            pl.BlockSpec((A_TILE, C), lambda i: (i, 0)),
            pl.BlockSpec((A_TILE, M, BD), lambda i: (i, 0, 0)),
            pl.BlockSpec((BD, C), lambda i: (0, 0)),
            pl.BlockSpec((8, C), lambda i: (0, 0)),
        ],
        out_specs=[
            pl.BlockSpec((A_TILE, AD), lambda i: (i, 0)),
            pl.BlockSpec((8, AD), lambda i: (0, 0)),
        ],
        out_shape=[
            jax.ShapeDtypeStruct((N, AD), jnp.float32),
            jax.ShapeDtypeStruct((8, AD), jnp.float32),
        ],
    )(g2, p1, bond, w3t, sums)


def _final_body(atom_ref, s_ref, sums2_ref, out_ref):
    mean = sums2_ref[0:1, :] * (1.0 / N)
    var = sums2_ref[1:2, :] * (1.0 / N) - mean * mean
    rstd = lax.rsqrt(var + EPS)
    sn = (s_ref[...] - mean) * rstd
    out_ref[...] = _softplus(atom_ref[...] + sn)


def _final(atom, s, sums2):
    rows = 1000
    return pl.pallas_call(
        _final_body,
        grid=(N // rows,),
        in_specs=[
            pl.BlockSpec((rows, AD), lambda i: (i, 0)),
            pl.BlockSpec((rows, AD), lambda i: (i, 0)),
            pl.BlockSpec((8, AD), lambda i: (0, 0)),
        ],
        out_specs=pl.BlockSpec((rows, AD), lambda i: (i, 0)),
        out_shape=jax.ShapeDtypeStruct((N, AD), jnp.float32),
    )(atom, s, sums2)


def kernel(atom, bond, bond_idx, W, b):
    w1t = W[:, :AD].T.astype(jnp.float32)
    w2t = W[:, AD:2 * AD].T.astype(jnp.float32)
    w3t = W[:, 2 * AD:].T.astype(jnp.float32)
    b2 = b.reshape(1, C)

    p1, p2 = _project(atom, w1t, w2t, b2)
    g2 = _sc_gather(bond_idx.reshape(NM), p2)
    sums = _stats(g2, p1, bond, w3t)
    s, sums2 = _apply(g2, p1, bond, w3t, sums)
    return _final(atom, s, sums2)


# speedup vs baseline: 2.0879x; 1.1474x over previous
import functools

import jax
import jax.numpy as jnp
from jax import lax
from jax.experimental import pallas as pl
from jax.experimental.pallas import tpu as pltpu
from jax.experimental.pallas import tpu_sc as plsc

N = 10000
M = 32
AD = 128
BD = 16
C = 2 * AD
NM = N * M
EPS = 1e-5

A_TILE = 40
R_TILE = A_TILE * M
GRID_ROWS = N // A_TILE

SC_CORES = 2
SC_SUBCORES = 16
NW = SC_CORES * SC_SUBCORES
ROWS_PER_W = NM // NW
CHUNK = 80
CHUNKS = ROWS_PER_W // CHUNK


def _softplus(x):
    return jnp.maximum(x, 0.0) + jnp.log1p(jnp.exp(-jnp.abs(x)))


def _pack_bf16_pair(lo, hi):
    ul = lax.bitcast_convert_type(lo, jnp.uint32)
    uh = lax.bitcast_convert_type(hi, jnp.uint32)
    bl = (ul + jnp.uint32(0x7FFF) + ((ul >> 16) & jnp.uint32(1))) >> 16
    bh = (uh + jnp.uint32(0x7FFF) + ((uh >> 16) & jnp.uint32(1))) >> 16
    return lax.bitcast_convert_type(bl | (bh << 16), jnp.int32)


def _unpack_bf16_pair(w):
    u = lax.bitcast_convert_type(w, jnp.uint32)
    lo = lax.bitcast_convert_type(u << 16, jnp.float32)
    hi = lax.bitcast_convert_type(u & jnp.uint32(0xFFFF0000), jnp.float32)
    return lo, hi


def _proj_body(atom_ref, w1t_ref, w2t_ref, b_ref, p1_ref, p2_ref):
    x = atom_ref[...]
    p1_ref[...] = (
        jnp.dot(x, w1t_ref[...], preferred_element_type=jnp.float32) + b_ref[...]
    )
    p2 = jnp.dot(x, w2t_ref[...], preferred_element_type=jnp.float32)
    p2_ref[...] = _pack_bf16_pair(p2[:, :AD], p2[:, AD:])


def _project(atom, w1t, w2t, b2):
    rows = 1000
    return pl.pallas_call(
        _proj_body,
        grid=(N // rows,),
        in_specs=[
            pl.BlockSpec((rows, AD), lambda i: (i, 0)),
            pl.BlockSpec((AD, C), lambda i: (0, 0)),
            pl.BlockSpec((AD, C), lambda i: (0, 0)),
            pl.BlockSpec((1, C), lambda i: (0, 0)),
        ],
        out_specs=[
            pl.BlockSpec((rows, C), lambda i: (i, 0)),
            pl.BlockSpec((rows, AD), lambda i: (i, 0)),
        ],
        out_shape=[
            jax.ShapeDtypeStruct((N, C), jnp.float32),
            jax.ShapeDtypeStruct((N, AD), jnp.int32),
        ],
    )(atom, w1t, w2t, b2)


def _gather_body(idx_hbm, p2_hbm, out_hbm, idx_v, rows0, rows1, sem0, sem1):
    wid = lax.axis_index("s") * SC_CORES + lax.axis_index("c")
    base = wid * ROWS_PER_W
    pltpu.sync_copy(idx_hbm.at[pl.ds(base, ROWS_PER_W)], idx_v)

    def gcopy(c, buf, sem):
        off = pl.multiple_of(c * CHUNK, 8)
        return pltpu.make_async_copy(
            p2_hbm.at[idx_v.at[pl.ds(off, CHUNK)]], buf, sem
        )

    def wback(c, buf):
        pltpu.sync_copy(buf, out_hbm.at[pl.ds(base + c * CHUNK, CHUNK)])

    gcopy(0, rows0, sem0).start()

    def step(i, carry):
        c0 = 2 * i
        gcopy(c0 + 1, rows1, sem1).start()
        gcopy(c0, rows0, sem0).wait()
        wback(c0, rows0)
        gcopy(c0 + 2, rows0, sem0).start()
        gcopy(c0 + 1, rows1, sem1).wait()
        wback(c0 + 1, rows1)
        return carry

    lax.fori_loop(0, (CHUNKS - 1) // 2, step, 0)
    gcopy(CHUNKS - 1, rows0, sem0).wait()
    wback(CHUNKS - 1, rows0)


def _sc_gather(flat_idx, p2):
    mesh = plsc.VectorSubcoreMesh(core_axis_name="c", subcore_axis_name="s")
    f = pl.kernel(
        _gather_body,
        out_type=jax.ShapeDtypeStruct((NM, AD), jnp.int32),
        mesh=mesh,
        scratch_types=[
            pltpu.VMEM((ROWS_PER_W,), jnp.int32),
            pltpu.VMEM((CHUNK, AD), jnp.int32),
            pltpu.VMEM((CHUNK, AD), jnp.int32),
            pltpu.SemaphoreType.DMA,
            pltpu.SemaphoreType.DMA,
        ],
    )
    return f(flat_idx, p2)


def _pre_activation(g2_ref, p1_ref, bond_ref, w3t_ref):
    p1 = p1_ref[...]
    p1rep = jnp.broadcast_to(p1[:, None, :], (A_TILE, M, C)).reshape(R_TILE, C)
    bw = jnp.dot(
        bond_ref[...].reshape(R_TILE, BD),
        w3t_ref[...],
        preferred_element_type=jnp.float32,
    )
    base = p1rep + bw
    lo, hi = _unpack_bf16_pair(g2_ref[...])
    return lo + base[:, :AD], hi + base[:, AD:]


def _stats_body(g2_ref, p1_ref, bond_ref, w3t_ref, sums_ref):
    gl, gh = _pre_activation(g2_ref, p1_ref, bond_ref, w3t_ref)

    @pl.when(pl.program_id(0) == 0)
    def _():
        sums_ref[...] = jnp.zeros((8, C), jnp.float32)

    sums_ref[0:1, :AD] += jnp.sum(gl, axis=0, keepdims=True)
    sums_ref[0:1, AD:] += jnp.sum(gh, axis=0, keepdims=True)
    sums_ref[1:2, :AD] += jnp.sum(gl * gl, axis=0, keepdims=True)
    sums_ref[1:2, AD:] += jnp.sum(gh * gh, axis=0, keepdims=True)


def _stats(g2, p1, bond, w3t):
    return pl.pallas_call(
        _stats_body,
        grid=(GRID_ROWS,),
        in_specs=[
            pl.BlockSpec((R_TILE, AD), lambda i: (i, 0)),
            pl.BlockSpec((A_TILE, C), lambda i: (i, 0)),
            pl.BlockSpec((A_TILE, M, BD), lambda i: (i, 0, 0)),
            pl.BlockSpec((BD, C), lambda i: (0, 0)),
        ],
        out_specs=pl.BlockSpec((8, C), lambda i: (0, 0)),
        out_shape=jax.ShapeDtypeStruct((8, C), jnp.float32),
    )(g2, p1, bond, w3t)


def _apply_body(g2_ref, p1_ref, bond_ref, w3t_ref, sums_ref, s_ref, sums2_ref):
    gl, gh = _pre_activation(g2_ref, p1_ref, bond_ref, w3t_ref)
    mean = sums_ref[0:1, :] * (1.0 / NM)
    var = sums_ref[1:2, :] * (1.0 / NM) - mean * mean
    rstd = lax.rsqrt(var + EPS)
    filt = jax.nn.sigmoid((gl - mean[:, :AD]) * rstd[:, :AD])
    core = _softplus((gh - mean[:, AD:]) * rstd[:, AD:])
    prod = filt * core
    s = jnp.sum(prod.reshape(A_TILE, M, AD), axis=1)
    s_ref[...] = s

    @pl.when(pl.program_id(0) == 0)
    def _():
        sums2_ref[...] = jnp.zeros((8, AD), jnp.float32)

    sums2_ref[0:1, :] += jnp.sum(s, axis=0, keepdims=True)
    sums2_ref[1:2, :] += jnp.sum(s * s, axis=0, keepdims=True)


def _apply(g2, p1, bond, w3t, sums):
    return pl.pallas_call(
        _apply_body,
        grid=(GRID_ROWS,),
        in_specs=[
            pl.BlockSpec((R_TILE, AD), lambda i: (i, 0)),
            pl.BlockSpec((A_TILE, C), lambda i: (i, 0)),
            pl.BlockSpec((A_TILE, M, BD), lambda i: (i, 0, 0)),
            pl.BlockSpec((BD, C), lambda i: (0, 0)),
            pl.BlockSpec((8, C), lambda i: (0, 0)),
        ],
        out_specs=[
            pl.BlockSpec((A_TILE, AD), lambda i: (i, 0)),
            pl.BlockSpec((8, AD), lambda i: (0, 0)),
        ],
        out_shape=[
            jax.ShapeDtypeStruct((N, AD), jnp.float32),
            jax.ShapeDtypeStruct((8, AD), jnp.float32),
        ],
    )(g2, p1, bond, w3t, sums)


def _final_body(atom_ref, s_ref, sums2_ref, out_ref):
    mean = sums2_ref[0:1, :] * (1.0 / N)
    var = sums2_ref[1:2, :] * (1.0 / N) - mean * mean
    rstd = lax.rsqrt(var + EPS)
    sn = (s_ref[...] - mean) * rstd
    out_ref[...] = _softplus(atom_ref[...] + sn)


def _final(atom, s, sums2):
    rows = 1000
    return pl.pallas_call(
        _final_body,
        grid=(N // rows,),
        in_specs=[
            pl.BlockSpec((rows, AD), lambda i: (i, 0)),
            pl.BlockSpec((rows, AD), lambda i: (i, 0)),
            pl.BlockSpec((8, AD), lambda i: (0, 0)),
        ],
        out_specs=pl.BlockSpec((rows, AD), lambda i: (i, 0)),
        out_shape=jax.ShapeDtypeStruct((N, AD), jnp.float32),
    )(atom, s, sums2)


def kernel(atom, bond, bond_idx, W, b):
    w1t = W[:, :AD].T.astype(jnp.float32)
    w2t = W[:, AD:2 * AD].T.astype(jnp.float32)
    w3t = W[:, 2 * AD:].T.astype(jnp.float32)
    b2 = b.reshape(1, C)

    p1, p2 = _project(atom, w1t, w2t, b2)
    g2 = _sc_gather(bond_idx.reshape(NM), p2)
    sums = _stats(g2, p1, bond, w3t)
    s, sums2 = _apply(g2, p1, bond, w3t, sums)
    return _final(atom, s, sums2)


# speedup vs baseline: 2.5936x; 1.2422x over previous
import functools

import jax
import jax.numpy as jnp
from jax import lax
from jax.experimental import pallas as pl
from jax.experimental.pallas import tpu as pltpu
from jax.experimental.pallas import tpu_sc as plsc

N = 10000
M = 32
AD = 128
BD = 16
C = 2 * AD
NM = N * M
EPS = 1e-5

A_TILE = 80
R_TILE = A_TILE * M
GRID_ROWS = N // A_TILE

SC_CORES = 2
SC_SUBCORES = 16
NW = SC_CORES * SC_SUBCORES
ROWS_PER_W = NM // NW
CHUNK = 80
CHUNKS = ROWS_PER_W // CHUNK


def _softplus(x):
    return jnp.maximum(x, 0.0) + jnp.log1p(jnp.exp(-jnp.abs(x)))


def _pack_bf16_pair(lo, hi):
    ul = lax.bitcast_convert_type(lo, jnp.uint32)
    uh = lax.bitcast_convert_type(hi, jnp.uint32)
    bl = (ul + jnp.uint32(0x7FFF) + ((ul >> 16) & jnp.uint32(1))) >> 16
    bh = (uh + jnp.uint32(0x7FFF) + ((uh >> 16) & jnp.uint32(1))) >> 16
    return lax.bitcast_convert_type(bl | (bh << 16), jnp.int32)


def _unpack_bf16_pair(w):
    u = lax.bitcast_convert_type(w, jnp.uint32)
    lo = lax.bitcast_convert_type(u << 16, jnp.float32)
    hi = lax.bitcast_convert_type(u, jnp.float32)
    return lo, hi


LOG2E = 1.4426950408889634


def _proj_body(atom_ref, w1t_ref, w2t_ref, b_ref, p1_ref, p2_ref):
    x = atom_ref[...]
    p1_ref[...] = (
        jnp.dot(x, w1t_ref[...], preferred_element_type=jnp.float32) + b_ref[...]
    )
    p2 = jnp.dot(x, w2t_ref[...], preferred_element_type=jnp.float32)
    p2_ref[...] = _pack_bf16_pair(p2[:, :AD], p2[:, AD:])


def _project(atom, w1t, w2t, b2):
    rows = 1000
    return pl.pallas_call(
        _proj_body,
        grid=(N // rows,),
        in_specs=[
            pl.BlockSpec((rows, AD), lambda i: (i, 0)),
            pl.BlockSpec((AD, C), lambda i: (0, 0)),
            pl.BlockSpec((AD, C), lambda i: (0, 0)),
            pl.BlockSpec((1, C), lambda i: (0, 0)),
        ],
        out_specs=[
            pl.BlockSpec((rows, C), lambda i: (i, 0)),
            pl.BlockSpec((rows, AD), lambda i: (i, 0)),
        ],
        out_shape=[
            jax.ShapeDtypeStruct((N, C), jnp.float32),
            jax.ShapeDtypeStruct((N, AD), jnp.int32),
        ],
    )(atom, w1t, w2t, b2)


def _gather_body(idx_hbm, p2_hbm, out_hbm, idx_v, rows0, rows1, sem0, sem1):
    wid = lax.axis_index("s") * SC_CORES + lax.axis_index("c")
    base = wid * ROWS_PER_W
    pltpu.sync_copy(idx_hbm.at[pl.ds(base, ROWS_PER_W)], idx_v)

    def gcopy(c, buf, sem):
        off = pl.multiple_of(c * CHUNK, 8)
        return pltpu.make_async_copy(
            p2_hbm.at[idx_v.at[pl.ds(off, CHUNK)]], buf, sem
        )

    def wback(c, buf):
        pltpu.sync_copy(buf, out_hbm.at[pl.ds(base + c * CHUNK, CHUNK)])

    gcopy(0, rows0, sem0).start()

    def step(i, carry):
        c0 = 2 * i
        gcopy(c0 + 1, rows1, sem1).start()
        gcopy(c0, rows0, sem0).wait()
        wback(c0, rows0)
        gcopy(c0 + 2, rows0, sem0).start()
        gcopy(c0 + 1, rows1, sem1).wait()
        wback(c0 + 1, rows1)
        return carry

    lax.fori_loop(0, (CHUNKS - 1) // 2, step, 0)
    gcopy(CHUNKS - 1, rows0, sem0).wait()
    wback(CHUNKS - 1, rows0)


def _sc_gather(flat_idx, p2):
    mesh = plsc.VectorSubcoreMesh(core_axis_name="c", subcore_axis_name="s")
    f = pl.kernel(
        _gather_body,
        out_type=jax.ShapeDtypeStruct((NM, AD), jnp.int32),
        mesh=mesh,
        scratch_types=[
            pltpu.VMEM((ROWS_PER_W,), jnp.int32),
            pltpu.VMEM((CHUNK, AD), jnp.int32),
            pltpu.VMEM((CHUNK, AD), jnp.int32),
            pltpu.SemaphoreType.DMA,
            pltpu.SemaphoreType.DMA,
        ],
    )
    return f(flat_idx, p2)


def _pre_activation(g2_ref, p1_ref, bond_ref, w3t_ref):
    p1 = p1_ref[...]
    p1rep = jnp.broadcast_to(p1[:, None, :], (A_TILE, M, C)).reshape(R_TILE, C)
    bw = jnp.dot(
        bond_ref[...].reshape(R_TILE, BD),
        w3t_ref[...],
        preferred_element_type=jnp.float32,
    )
    base = p1rep + bw
    lo, hi = _unpack_bf16_pair(g2_ref[...])
    return lo + base[:, :AD], hi + base[:, AD:]


def _stats_body(g2_ref, p1_ref, bond_ref, w3t_ref, sums_ref):
    gl, gh = _pre_activation(g2_ref, p1_ref, bond_ref, w3t_ref)

    @pl.when(pl.program_id(0) == 0)
    def _():
        sums_ref[...] = jnp.zeros((8, C), jnp.float32)

    sums_ref[0:1, :AD] += jnp.sum(gl, axis=0, keepdims=True)
    sums_ref[0:1, AD:] += jnp.sum(gh, axis=0, keepdims=True)
    sums_ref[1:2, :AD] += jnp.sum(gl * gl, axis=0, keepdims=True)
    sums_ref[1:2, AD:] += jnp.sum(gh * gh, axis=0, keepdims=True)


def _stats(g2, p1, bond, w3t):
    return pl.pallas_call(
        _stats_body,
        grid=(GRID_ROWS,),
        in_specs=[
            pl.BlockSpec((R_TILE, AD), lambda i: (i, 0)),
            pl.BlockSpec((A_TILE, C), lambda i: (i, 0)),
            pl.BlockSpec((A_TILE, M, BD), lambda i: (i, 0, 0)),
            pl.BlockSpec((BD, C), lambda i: (0, 0)),
        ],
        out_specs=pl.BlockSpec((8, C), lambda i: (0, 0)),
        out_shape=jax.ShapeDtypeStruct((8, C), jnp.float32),
    )(g2, p1, bond, w3t)


def _apply_body(g2_ref, p1_ref, bond_ref, w3t_ref, sums_ref, s_ref, sums2_ref):
    gl, gh = _pre_activation(g2_ref, p1_ref, bond_ref, w3t_ref)
    mean = sums_ref[0:1, :] * (1.0 / NM)
    var = sums_ref[1:2, :] * (1.0 / NM) - mean * mean
    rstd = lax.rsqrt(var + EPS)
    rl = rstd[:, :AD]
    af = rl * (-LOG2E)
    cf = mean[:, :AD] * rl * LOG2E
    filt = 1.0 / (1.0 + jnp.exp2(gl * af + cf))
    rh = rstd[:, AD:]
    y = gh * rh - mean[:, AD:] * rh
    t = jnp.exp2(jnp.abs(y) * (-LOG2E))
    core = jnp.maximum(y, 0.0) + jnp.log1p(t)
    prod = filt * core
    s = jnp.sum(prod.reshape(A_TILE, M, AD), axis=1)
    s_ref[...] = s

    @pl.when(pl.program_id(0) == 0)
    def _():
        sums2_ref[...] = jnp.zeros((8, AD), jnp.float32)

    sums2_ref[0:1, :] += jnp.sum(s, axis=0, keepdims=True)
    sums2_ref[1:2, :] += jnp.sum(s * s, axis=0, keepdims=True)


def _apply(g2, p1, bond, w3t, sums):
    return pl.pallas_call(
        _apply_body,
        grid=(GRID_ROWS,),
        in_specs=[
            pl.BlockSpec((R_TILE, AD), lambda i: (i, 0)),
            pl.BlockSpec((A_TILE, C), lambda i: (i, 0)),
            pl.BlockSpec((A_TILE, M, BD), lambda i: (i, 0, 0)),
            pl.BlockSpec((BD, C), lambda i: (0, 0)),
            pl.BlockSpec((8, C), lambda i: (0, 0)),
        ],
        out_specs=[
            pl.BlockSpec((A_TILE, AD), lambda i: (i, 0)),
            pl.BlockSpec((8, AD), lambda i: (0, 0)),
        ],
        out_shape=[
            jax.ShapeDtypeStruct((N, AD), jnp.float32),
            jax.ShapeDtypeStruct((8, AD), jnp.float32),
        ],
    )(g2, p1, bond, w3t, sums)


def _final_body(atom_ref, s_ref, sums2_ref, out_ref):
    mean = sums2_ref[0:1, :] * (1.0 / N)
    var = sums2_ref[1:2, :] * (1.0 / N) - mean * mean
    rstd = lax.rsqrt(var + EPS)
    sn = (s_ref[...] - mean) * rstd
    out_ref[...] = _softplus(atom_ref[...] + sn)


def _final(atom, s, sums2):
    rows = 1000
    return pl.pallas_call(
        _final_body,
        grid=(N // rows,),
        in_specs=[
            pl.BlockSpec((rows, AD), lambda i: (i, 0)),
            pl.BlockSpec((rows, AD), lambda i: (i, 0)),
            pl.BlockSpec((8, AD), lambda i: (0, 0)),
        ],
        out_specs=pl.BlockSpec((rows, AD), lambda i: (i, 0)),
        out_shape=jax.ShapeDtypeStruct((N, AD), jnp.float32),
    )(atom, s, sums2)


def kernel(atom, bond, bond_idx, W, b):
    w1t = W[:, :AD].T.astype(jnp.float32)
    w2t = W[:, AD:2 * AD].T.astype(jnp.float32)
    w3t = W[:, 2 * AD:].T.astype(jnp.float32)
    b2 = b.reshape(1, C)

    p1, p2 = _project(atom, w1t, w2t, b2)
    g2 = _sc_gather(bond_idx.reshape(NM), p2)
    sums = _stats(g2, p1, bond, w3t)
    s, sums2 = _apply(g2, p1, bond, w3t, sums)
    return _final(atom, s, sums2)


# speedup vs baseline: 2.8594x; 1.1025x over previous
import jax
import jax.numpy as jnp
from jax import lax
from jax.experimental import pallas as pl
from jax.experimental.pallas import tpu as pltpu
from jax.experimental.pallas import tpu_sc as plsc

N = 10000
M = 32
AD = 128
BD = 16
C = 2 * AD
NM = N * M
EPS = 1e-5
LOG2E = 1.4426950408889634

HALVES = 2
N_H = N // HALVES
NM_H = NM // HALVES

A_TILE = 200
R_TILE = A_TILE * M
GRID_H = N_H // A_TILE

SC_CORES = 2
SC_SUBCORES = 16
NW = SC_CORES * SC_SUBCORES
ROWS_PER_W = NM_H // NW
CHUNK = 40
CHUNKS = ROWS_PER_W // CHUNK


def _pack_bf16_pair(lo, hi):
    ul = lax.bitcast_convert_type(lo, jnp.uint32)
    uh = lax.bitcast_convert_type(hi, jnp.uint32)
    bl = (ul + jnp.uint32(0x7FFF) + ((ul >> 16) & jnp.uint32(1))) >> 16
    bh = (uh + jnp.uint32(0x7FFF) + ((uh >> 16) & jnp.uint32(1))) >> 16
    return lax.bitcast_convert_type(bl | (bh << 16), jnp.int32)


def _unpack_bf16_pair(w):
    u = lax.bitcast_convert_type(w, jnp.uint32)
    lo = lax.bitcast_convert_type(u << 16, jnp.float32)
    hi = lax.bitcast_convert_type(u, jnp.float32)
    return lo, hi


def _proj_body(atom_ref, w1t_ref, w2t_ref, b_ref, p1_ref, p2_ref):
    x = atom_ref[...]
    p1_ref[...] = (
        jnp.dot(x, w1t_ref[...], preferred_element_type=jnp.float32) + b_ref[...]
    )
    p2 = jnp.dot(x, w2t_ref[...], preferred_element_type=jnp.float32)
    p2_ref[...] = _pack_bf16_pair(p2[:, :AD], p2[:, AD:])


def _project(atom, w1t, w2t, b2):
    rows = 1000
    return pl.pallas_call(
        _proj_body,
        grid=(N // rows,),
        in_specs=[
            pl.BlockSpec((rows, AD), lambda i: (i, 0)),
            pl.BlockSpec((AD, C), lambda i: (0, 0)),
            pl.BlockSpec((AD, C), lambda i: (0, 0)),
            pl.BlockSpec((1, C), lambda i: (0, 0)),
        ],
        out_specs=[
            pl.BlockSpec((rows, C), lambda i: (i, 0)),
            pl.BlockSpec((rows, AD), lambda i: (i, 0)),
        ],
        out_shape=[
            jax.ShapeDtypeStruct((N, C), jnp.float32),
            jax.ShapeDtypeStruct((N, AD), jnp.int32),
        ],
    )(atom, w1t, w2t, b2)


def _gather_body(idx_hbm, p2_hbm, out_hbm, idx_v, rows0, rows1, sem0, sem1):
    wid = lax.axis_index("s") * SC_CORES + lax.axis_index("c")
    base = wid * ROWS_PER_W
    pltpu.sync_copy(idx_hbm.at[pl.ds(base, ROWS_PER_W)], idx_v)

    def gcopy(c, buf, sem):
        off = pl.multiple_of(c * CHUNK, 8)
        return pltpu.make_async_copy(
            p2_hbm.at[idx_v.at[pl.ds(off, CHUNK)]], buf, sem
        )

    def wback(c, buf):
        pltpu.sync_copy(buf, out_hbm.at[pl.ds(base + c * CHUNK, CHUNK)])

    gcopy(0, rows0, sem0).start()

    def step(i, carry):
        c0 = 2 * i
        gcopy(c0 + 1, rows1, sem1).start()
        gcopy(c0, rows0, sem0).wait()
        wback(c0, rows0)
        gcopy(c0 + 2, rows0, sem0).start()
        gcopy(c0 + 1, rows1, sem1).wait()
        wback(c0 + 1, rows1)
        return carry

    lax.fori_loop(0, (CHUNKS - 1) // 2, step, 0)
    gcopy(CHUNKS - 1, rows0, sem0).wait()
    wback(CHUNKS - 1, rows0)


def _sc_gather(flat_idx_half, p2):
    mesh = plsc.VectorSubcoreMesh(core_axis_name="c", subcore_axis_name="s")
    f = pl.kernel(
        _gather_body,
        out_type=jax.ShapeDtypeStruct((NM_H, AD), jnp.int32),
        mesh=mesh,
        scratch_types=[
            pltpu.VMEM((ROWS_PER_W,), jnp.int32),
            pltpu.VMEM((CHUNK, AD), jnp.int32),
            pltpu.VMEM((CHUNK, AD), jnp.int32),
            pltpu.SemaphoreType.DMA,
            pltpu.SemaphoreType.DMA,
        ],
    )
    return f(flat_idx_half, p2)


def _pre_activation(g2_ref, p1_ref, bond_ref, w3t_ref):
    p1 = p1_ref[...]
    p1rep = jnp.broadcast_to(p1[:, None, :], (A_TILE, M, C)).reshape(R_TILE, C)
    bw = jnp.dot(
        bond_ref[...].reshape(R_TILE, BD),
        w3t_ref[...],
        preferred_element_type=jnp.float32,
    )
    base = p1rep + bw
    lo, hi = _unpack_bf16_pair(g2_ref[...])
    return lo + base[:, :AD], hi + base[:, AD:]


def _stats_body(g2_ref, p1_ref, bond_ref, w3t_ref, sums_ref):
    gl, gh = _pre_activation(g2_ref, p1_ref, bond_ref, w3t_ref)

    @pl.when(pl.program_id(0) == 0)
    def _():
        sums_ref[...] = jnp.zeros((8, C), jnp.float32)

    sums_ref[0:1, :AD] += jnp.sum(gl, axis=0, keepdims=True)
    sums_ref[0:1, AD:] += jnp.sum(gh, axis=0, keepdims=True)
    sums_ref[1:2, :AD] += jnp.sum(gl * gl, axis=0, keepdims=True)
    sums_ref[1:2, AD:] += jnp.sum(gh * gh, axis=0, keepdims=True)


def _stats(g2h, p1, bond, w3t, half):
    a0 = half * GRID_H
    return pl.pallas_call(
        _stats_body,
        grid=(GRID_H,),
        in_specs=[
            pl.BlockSpec((R_TILE, AD), lambda i: (i, 0)),
            pl.BlockSpec((A_TILE, C), lambda i: (i + a0, 0)),
            pl.BlockSpec((A_TILE, M, BD), lambda i: (i + a0, 0, 0)),
            pl.BlockSpec((BD, C), lambda i: (0, 0)),
        ],
        out_specs=pl.BlockSpec((8, C), lambda i: (0, 0)),
        out_shape=jax.ShapeDtypeStruct((8, C), jnp.float32),
    )(g2h, p1, bond, w3t)


def _apply_body(g2_ref, p1_ref, bond_ref, w3t_ref, sa_ref, sb_ref, s_ref, sums2_ref):
    gl, gh = _pre_activation(g2_ref, p1_ref, bond_ref, w3t_ref)
    sums = sa_ref[...] + sb_ref[...]
    mean = sums[0:1, :] * (1.0 / NM)
    var = sums[1:2, :] * (1.0 / NM) - mean * mean
    rstd = lax.rsqrt(var + EPS)
    rl = rstd[:, :AD]
    af = rl * (-LOG2E)
    cf = mean[:, :AD] * rl * LOG2E
    filt = 1.0 / (1.0 + jnp.exp2(gl * af + cf))
    rh = rstd[:, AD:]
    y = gh * rh - mean[:, AD:] * rh
    t = jnp.exp2(jnp.abs(y) * (-LOG2E))
    core = jnp.maximum(y, 0.0) + jnp.log1p(t)
    prod = filt * core
    s = jnp.sum(prod.reshape(A_TILE, M, AD), axis=1)
    s_ref[...] = s

    @pl.when(pl.program_id(0) == 0)
    def _():
        sums2_ref[...] = jnp.zeros((8, AD), jnp.float32)

    sums2_ref[0:1, :] += jnp.sum(s, axis=0, keepdims=True)
    sums2_ref[1:2, :] += jnp.sum(s * s, axis=0, keepdims=True)


def _apply(g2h, p1, bond, w3t, sums_a, sums_b, half):
    a0 = half * GRID_H
    return pl.pallas_call(
        _apply_body,
        grid=(GRID_H,),
        in_specs=[
            pl.BlockSpec((R_TILE, AD), lambda i: (i, 0)),
            pl.BlockSpec((A_TILE, C), lambda i: (i + a0, 0)),
            pl.BlockSpec((A_TILE, M, BD), lambda i: (i + a0, 0, 0)),
            pl.BlockSpec((BD, C), lambda i: (0, 0)),
            pl.BlockSpec((8, C), lambda i: (0, 0)),
            pl.BlockSpec((8, C), lambda i: (0, 0)),
        ],
        out_specs=[
            pl.BlockSpec((A_TILE, AD), lambda i: (i, 0)),
            pl.BlockSpec((8, AD), lambda i: (0, 0)),
        ],
        out_shape=[
            jax.ShapeDtypeStruct((N_H, AD), jnp.float32),
            jax.ShapeDtypeStruct((8, AD), jnp.float32),
        ],
    )(g2h, p1, bond, w3t, sums_a, sums_b)


def _final_body(atom_ref, s_ref, s2a_ref, s2b_ref, out_ref):
    sums2 = s2a_ref[...] + s2b_ref[...]
    mean = sums2[0:1, :] * (1.0 / N)
    var = sums2[1:2, :] * (1.0 / N) - mean * mean
    rstd = lax.rsqrt(var + EPS)
    sn = (s_ref[...] - mean) * rstd
    x = atom_ref[...] + sn
    out_ref[...] = jnp.maximum(x, 0.0) + jnp.log1p(jnp.exp2(jnp.abs(x) * (-LOG2E)))


def _final(atom, s_h, sums2_a, sums2_b, half):
    rows = 1000
    a0 = half * (N_H // rows)
    return pl.pallas_call(
        _final_body,
        grid=(N_H // rows,),
        in_specs=[
            pl.BlockSpec((rows, AD), lambda i: (i + a0, 0)),
            pl.BlockSpec((rows, AD), lambda i: (i, 0)),
            pl.BlockSpec((8, AD), lambda i: (0, 0)),
            pl.BlockSpec((8, AD), lambda i: (0, 0)),
        ],
        out_specs=pl.BlockSpec((rows, AD), lambda i: (i, 0)),
        out_shape=jax.ShapeDtypeStruct((N_H, AD), jnp.float32),
    )(atom, s_h, sums2_a, sums2_b)


def kernel(atom, bond, bond_idx, W, b):
    w1t = W[:, :AD].T.astype(jnp.float32)
    w2t = W[:, AD:2 * AD].T.astype(jnp.float32)
    w3t = W[:, 2 * AD:].T.astype(jnp.float32)
    b2 = b.reshape(1, C)

    p1, p2 = _project(atom, w1t, w2t, b2)
    idx2 = bond_idx.reshape(HALVES, NM_H)
    g2a = _sc_gather(idx2[0], p2)
    g2b = _sc_gather(idx2[1], p2)
    sums_a = _stats(g2a, p1, bond, w3t, 0)
    sums_b = _stats(g2b, p1, bond, w3t, 1)
    s_a, sums2_a = _apply(g2a, p1, bond, w3t, sums_a, sums_b, 0)
    s_b, sums2_b = _apply(g2b, p1, bond, w3t, sums_a, sums_b, 1)
    out_a = _final(atom, s_a, sums2_a, sums2_b, 0)
    out_b = _final(atom, s_b, sums2_a, sums2_b, 1)
    return jnp.concatenate([out_a, out_b], axis=0)


# speedup vs baseline: 2.9095x; 1.0175x over previous
import jax
import jax.numpy as jnp
from jax import lax
from jax.experimental import pallas as pl
from jax.experimental.pallas import tpu as pltpu
from jax.experimental.pallas import tpu_sc as plsc

N = 10000
M = 32
AD = 128
BD = 16
C = 2 * AD
NM = N * M
EPS = 1e-5
LOG2E = 1.4426950408889634

HALVES = 2
N_H = N // HALVES
NM_H = NM // HALVES

A_TILE = 200
R_TILE = A_TILE * M
GRID_H = N_H // A_TILE

SC_CORES = 2
SC_SUBCORES = 16
NW = SC_CORES * SC_SUBCORES
ROWS_PER_W = NM_H // NW
CHUNK = 40
CHUNKS = ROWS_PER_W // CHUNK


def _pack_bf16_pair(lo, hi):
    ul = lax.bitcast_convert_type(lo, jnp.uint32)
    uh = lax.bitcast_convert_type(hi, jnp.uint32)
    bl = (ul + jnp.uint32(0x7FFF) + ((ul >> 16) & jnp.uint32(1))) >> 16
    bh = (uh + jnp.uint32(0x7FFF) + ((uh >> 16) & jnp.uint32(1))) >> 16
    return lax.bitcast_convert_type(bl | (bh << 16), jnp.int32)


def _unpack_bf16_pair(w):
    u = lax.bitcast_convert_type(w, jnp.uint32)
    lo = lax.bitcast_convert_type(u << 16, jnp.float32)
    hi = lax.bitcast_convert_type(u, jnp.float32)
    return lo, hi


def _proj_body(atom_ref, w1t_ref, w2t_ref, b_ref, p1_ref, p2_ref):
    x = atom_ref[...]
    p1_ref[...] = (
        jnp.dot(x, w1t_ref[...], preferred_element_type=jnp.float32) + b_ref[...]
    )
    p2 = jnp.dot(x, w2t_ref[...], preferred_element_type=jnp.float32)
    p2_ref[...] = _pack_bf16_pair(p2[:, :AD], p2[:, AD:])


def _project(atom, w1t, w2t, b2):
    rows = 1000
    return pl.pallas_call(
        _proj_body,
        grid=(N // rows,),
        in_specs=[
            pl.BlockSpec((rows, AD), lambda i: (i, 0)),
            pl.BlockSpec((AD, C), lambda i: (0, 0)),
            pl.BlockSpec((AD, C), lambda i: (0, 0)),
            pl.BlockSpec((1, C), lambda i: (0, 0)),
        ],
        out_specs=[
            pl.BlockSpec((rows, C), lambda i: (i, 0)),
            pl.BlockSpec((rows, AD), lambda i: (i, 0)),
        ],
        out_shape=[
            jax.ShapeDtypeStruct((N, C), jnp.float32),
            jax.ShapeDtypeStruct((N, AD), jnp.int32),
        ],
    )(atom, w1t, w2t, b2)


def _gather_body(idx_hbm, p2_hbm, out_hbm, idx_v, rows0, rows1, sem0, sem1):
    wid = lax.axis_index("s") * SC_CORES + lax.axis_index("c")
    base = wid * ROWS_PER_W
    pltpu.sync_copy(idx_hbm.at[pl.ds(base, ROWS_PER_W)], idx_v)

    def gcopy(c, buf, sem):
        off = pl.multiple_of(c * CHUNK, 8)
        return pltpu.make_async_copy(
            p2_hbm.at[idx_v.at[pl.ds(off, CHUNK)]], buf, sem
        )

    def wback(c, buf):
        pltpu.sync_copy(buf, out_hbm.at[pl.ds(base + c * CHUNK, CHUNK)])

    gcopy(0, rows0, sem0).start()

    def step(i, carry):
        c0 = 2 * i
        gcopy(c0 + 1, rows1, sem1).start()
        gcopy(c0, rows0, sem0).wait()
        wback(c0, rows0)
        gcopy(c0 + 2, rows0, sem0).start()
        gcopy(c0 + 1, rows1, sem1).wait()
        wback(c0 + 1, rows1)
        return carry

    lax.fori_loop(0, (CHUNKS - 1) // 2, step, 0)
    gcopy(CHUNKS - 1, rows0, sem0).wait()
    wback(CHUNKS - 1, rows0)


def _sc_gather(flat_idx_half, p2):
    mesh = plsc.VectorSubcoreMesh(core_axis_name="c", subcore_axis_name="s")
    f = pl.kernel(
        _gather_body,
        out_type=jax.ShapeDtypeStruct((NM_H, AD), jnp.int32),
        mesh=mesh,
        scratch_types=[
            pltpu.VMEM((ROWS_PER_W,), jnp.int32),
            pltpu.VMEM((CHUNK, AD), jnp.int32),
            pltpu.VMEM((CHUNK, AD), jnp.int32),
            pltpu.SemaphoreType.DMA,
            pltpu.SemaphoreType.DMA,
        ],
    )
    return f(flat_idx_half, p2)


def _pre_activation(g2_ref, p1_ref, bond_ref, w3t_ref):
    p1 = p1_ref[...]
    p1rep = jnp.broadcast_to(p1[:, None, :], (A_TILE, M, C)).reshape(R_TILE, C)
    bw = jnp.dot(
        bond_ref[...],
        w3t_ref[...],
        preferred_element_type=jnp.float32,
    )
    base = p1rep + bw
    lo, hi = _unpack_bf16_pair(g2_ref[...])
    return lo + base[:, :AD], hi + base[:, AD:]


def _stats_body(g2_ref, p1_ref, bond_ref, w3t_ref, sums_ref):
    gl, gh = _pre_activation(g2_ref, p1_ref, bond_ref, w3t_ref)

    @pl.when(pl.program_id(0) == 0)
    def _():
        sums_ref[...] = jnp.zeros((8, C), jnp.float32)

    sums_ref[0:1, :AD] += jnp.sum(gl, axis=0, keepdims=True)
    sums_ref[0:1, AD:] += jnp.sum(gh, axis=0, keepdims=True)
    sums_ref[1:2, :AD] += jnp.sum(gl * gl, axis=0, keepdims=True)
    sums_ref[1:2, AD:] += jnp.sum(gh * gh, axis=0, keepdims=True)


def _stats(g2h, p1, bond2, w3t, half):
    a0 = half * GRID_H
    return pl.pallas_call(
        _stats_body,
        grid=(GRID_H,),
        in_specs=[
            pl.BlockSpec((R_TILE, AD), lambda i: (i, 0)),
            pl.BlockSpec((A_TILE, C), lambda i: (i + a0, 0)),
            pl.BlockSpec((R_TILE, BD), lambda i: (i + a0, 0)),
            pl.BlockSpec((BD, C), lambda i: (0, 0)),
        ],
        out_specs=pl.BlockSpec((8, C), lambda i: (0, 0)),
        out_shape=jax.ShapeDtypeStruct((8, C), jnp.float32),
    )(g2h, p1, bond2, w3t)


def _apply_body(g2_ref, p1_ref, bond_ref, w3t_ref, sa_ref, sb_ref, s_ref, sums2_ref):
    gl, gh = _pre_activation(g2_ref, p1_ref, bond_ref, w3t_ref)
    sums = sa_ref[...] + sb_ref[...]
    mean = sums[0:1, :] * (1.0 / NM)
    var = sums[1:2, :] * (1.0 / NM) - mean * mean
    rstd = lax.rsqrt(var + EPS)
    rl = rstd[:, :AD]
    af = rl * (-LOG2E)
    cf = mean[:, :AD] * rl * LOG2E
    filt = 1.0 / (1.0 + jnp.exp2(gl * af + cf))
    rh = rstd[:, AD:]
    y = gh * rh - mean[:, AD:] * rh
    t = jnp.exp2(jnp.abs(y) * (-LOG2E))
    core = jnp.maximum(y, 0.0) + jnp.log1p(t)
    prod = filt * core
    s = jnp.sum(prod.reshape(A_TILE, M, AD), axis=1)
    s_ref[...] = s

    @pl.when(pl.program_id(0) == 0)
    def _():
        sums2_ref[...] = jnp.zeros((8, AD), jnp.float32)

    sums2_ref[0:1, :] += jnp.sum(s, axis=0, keepdims=True)
    sums2_ref[1:2, :] += jnp.sum(s * s, axis=0, keepdims=True)


def _apply(g2h, p1, bond2, w3t, sums_a, sums_b, half):
    a0 = half * GRID_H
    return pl.pallas_call(
        _apply_body,
        grid=(GRID_H,),
        in_specs=[
            pl.BlockSpec((R_TILE, AD), lambda i: (i, 0)),
            pl.BlockSpec((A_TILE, C), lambda i: (i + a0, 0)),
            pl.BlockSpec((R_TILE, BD), lambda i: (i + a0, 0)),
            pl.BlockSpec((BD, C), lambda i: (0, 0)),
            pl.BlockSpec((8, C), lambda i: (0, 0)),
            pl.BlockSpec((8, C), lambda i: (0, 0)),
        ],
        out_specs=[
            pl.BlockSpec((A_TILE, AD), lambda i: (i, 0)),
            pl.BlockSpec((8, AD), lambda i: (0, 0)),
        ],
        out_shape=[
            jax.ShapeDtypeStruct((N_H, AD), jnp.float32),
            jax.ShapeDtypeStruct((8, AD), jnp.float32),
        ],
    )(g2h, p1, bond2, w3t, sums_a, sums_b)


def _final_body(atom_ref, s_ref, s2a_ref, s2b_ref, out_ref):
    sums2 = s2a_ref[...] + s2b_ref[...]
    mean = sums2[0:1, :] * (1.0 / N)
    var = sums2[1:2, :] * (1.0 / N) - mean * mean
    rstd = lax.rsqrt(var + EPS)
    sn = (s_ref[...] - mean) * rstd
    x = atom_ref[...] + sn
    out_ref[...] = jnp.maximum(x, 0.0) + jnp.log1p(jnp.exp2(jnp.abs(x) * (-LOG2E)))


def _final(atom, s_h, sums2_a, sums2_b, half):
    rows = 1000
    a0 = half * (N_H // rows)
    return pl.pallas_call(
        _final_body,
        grid=(N_H // rows,),
        in_specs=[
            pl.BlockSpec((rows, AD), lambda i: (i + a0, 0)),
            pl.BlockSpec((rows, AD), lambda i: (i, 0)),
            pl.BlockSpec((8, AD), lambda i: (0, 0)),
            pl.BlockSpec((8, AD), lambda i: (0, 0)),
        ],
        out_specs=pl.BlockSpec((rows, AD), lambda i: (i, 0)),
        out_shape=jax.ShapeDtypeStruct((N_H, AD), jnp.float32),
    )(atom, s_h, sums2_a, sums2_b)


def kernel(atom, bond, bond_idx, W, b):
    w1t = W[:, :AD].T.astype(jnp.float32)
    w2t = W[:, AD:2 * AD].T.astype(jnp.float32)
    w3t = W[:, 2 * AD:].T.astype(jnp.float32)
    b2 = b.reshape(1, C)

    p1, p2 = _project(atom, w1t, w2t, b2)
    bond2 = bond.reshape(NM, BD)
    idx2 = bond_idx.reshape(HALVES, NM_H)
    g2a = _sc_gather(idx2[0], p2)
    g2b = _sc_gather(idx2[1], p2)
    sums_a = _stats(g2a, p1, bond2, w3t, 0)
    sums_b = _stats(g2b, p1, bond2, w3t, 1)
    s_a, sums2_a = _apply(g2a, p1, bond2, w3t, sums_a, sums_b, 0)
    s_b, sums2_b = _apply(g2b, p1, bond2, w3t, sums_a, sums_b, 1)
    out_a = _final(atom, s_a, sums2_a, sums2_b, 0)
    out_b = _final(atom, s_b, sums2_a, sums2_b, 1)
    return jnp.concatenate([out_a, out_b], axis=0)


# speedup vs baseline: 3.1684x; 1.0890x over previous
import jax
import jax.numpy as jnp
from jax import lax
from jax.experimental import pallas as pl
from jax.experimental.pallas import tpu as pltpu
from jax.experimental.pallas import tpu_sc as plsc

N = 10000
M = 32
AD = 128
BD = 16
C = 2 * AD
NM = N * M
EPS = 1e-5
LOG2E = 1.4426950408889634

HALVES = 2
N_H = N // HALVES
NM_H = NM // HALVES

A_TILE = 200
R_TILE = A_TILE * M
GRID_H = N_H // A_TILE

SC_CORES = 2
SC_SUBCORES = 16
NW = SC_CORES * SC_SUBCORES
ROWS_PER_W = NM_H // NW
CHUNK = 40
CHUNKS = ROWS_PER_W // CHUNK


def _pack_bf16_pair(lo, hi):
    ul = lax.bitcast_convert_type(lo, jnp.uint32)
    uh = lax.bitcast_convert_type(hi, jnp.uint32)
    bl = (ul + jnp.uint32(0x7FFF) + ((ul >> 16) & jnp.uint32(1))) >> 16
    bh = (uh + jnp.uint32(0x7FFF) + ((uh >> 16) & jnp.uint32(1))) >> 16
    return lax.bitcast_convert_type(bl | (bh << 16), jnp.int32)


def _unpack_bf16_pair(w):
    u = lax.bitcast_convert_type(w, jnp.uint32)
    lo = lax.bitcast_convert_type(u << 16, jnp.float32)
    hi = lax.bitcast_convert_type(u, jnp.float32)
    return lo, hi


def _proj_body(atom_ref, w1t_ref, w2t_ref, b_ref, p1_ref, p2_ref):
    x = atom_ref[...]
    p1_ref[...] = (
        jnp.dot(x, w1t_ref[...], preferred_element_type=jnp.float32) + b_ref[...]
    )
    p2 = jnp.dot(x, w2t_ref[...], preferred_element_type=jnp.float32)
    p2_ref[...] = _pack_bf16_pair(p2[:, :AD], p2[:, AD:])


def _project(atom, w1t, w2t, b2):
    rows = 1000
    return pl.pallas_call(
        _proj_body,
        grid=(N // rows,),
        in_specs=[
            pl.BlockSpec((rows, AD), lambda i: (i, 0)),
            pl.BlockSpec((AD, C), lambda i: (0, 0)),
            pl.BlockSpec((AD, C), lambda i: (0, 0)),
            pl.BlockSpec((1, C), lambda i: (0, 0)),
        ],
        out_specs=[
            pl.BlockSpec((rows, C), lambda i: (i, 0)),
            pl.BlockSpec((rows, AD), lambda i: (i, 0)),
        ],
        out_shape=[
            jax.ShapeDtypeStruct((N, C), jnp.float32),
            jax.ShapeDtypeStruct((N, AD), jnp.int32),
        ],
    )(atom, w1t, w2t, b2)


def _gather_body(idx_hbm, p2_hbm, out_hbm, idx_v,
                 b0, b1, b2, b3, g0, g1, g2, g3, w0, w1, w2, w3):
    wid = lax.axis_index("s") * SC_CORES + lax.axis_index("c")
    base = wid * ROWS_PER_W
    bufs = (b0, b1, b2, b3)
    gsem = (g0, g1, g2, g3)
    wsem = (w0, w1, w2, w3)
    pltpu.sync_copy(idx_hbm.at[pl.ds(base, ROWS_PER_W)], idx_v)

    def gd(c, k):
        off = pl.multiple_of(c * CHUNK, 8)
        return pltpu.make_async_copy(
            p2_hbm.at[idx_v.at[pl.ds(off, CHUNK)]], bufs[k], gsem[k]
        )

    def wd(c, k):
        return pltpu.make_async_copy(
            bufs[k], out_hbm.at[pl.ds(base + c * CHUNK, CHUNK)], wsem[k]
        )

    def chunk(c, k, wait_prev, start_next):
        gd(c, k).wait()
        wd(c, k).start()
        if wait_prev:
            wd(c - 2, (k - 2) % 4).wait()
        if start_next:
            gd(c + 2, (k + 2) % 4).start()

    gd(0, 0).start()
    gd(1, 1).start()
    chunk(0, 0, False, True)
    chunk(1, 1, False, True)
    chunk(2, 2, True, True)
    chunk(3, 3, True, True)

    def step(t, carry):
        c0 = 4 + 4 * t
        chunk(c0, 0, True, True)
        chunk(c0 + 1, 1, True, True)
        chunk(c0 + 2, 2, True, True)
        chunk(c0 + 3, 3, True, True)
        return carry

    lax.fori_loop(0, (CHUNKS - 9) // 4, step, 0)
    c0 = CHUNKS - 5
    chunk(c0, 0, True, True)
    chunk(c0 + 1, 1, True, True)
    chunk(c0 + 2, 2, True, True)
    chunk(c0 + 3, 3, True, False)
    chunk(c0 + 4, 0, True, False)
    wd(c0 + 3, 3).wait()
    wd(c0 + 4, 0).wait()


def _sc_gather(flat_idx_half, p2):
    mesh = plsc.VectorSubcoreMesh(core_axis_name="c", subcore_axis_name="s")
    f = pl.kernel(
        _gather_body,
        out_type=jax.ShapeDtypeStruct((NM_H, AD), jnp.int32),
        mesh=mesh,
        scratch_types=(
            [pltpu.VMEM((ROWS_PER_W,), jnp.int32)]
            + [pltpu.VMEM((CHUNK, AD), jnp.int32) for _ in range(4)]
            + [pltpu.SemaphoreType.DMA for _ in range(8)]
        ),
    )
    return f(flat_idx_half, p2)


def _pre_activation(g2_ref, p1_ref, bond_ref, w3t_ref):
    p1 = p1_ref[...]
    p1rep = jnp.broadcast_to(p1[:, None, :], (A_TILE, M, C)).reshape(R_TILE, C)
    bw = jnp.dot(
        bond_ref[...],
        w3t_ref[...],
        preferred_element_type=jnp.float32,
    )
    base = p1rep + bw
    lo, hi = _unpack_bf16_pair(g2_ref[...])
    return lo + base[:, :AD], hi + base[:, AD:]


def _stats_body(g2_ref, p1_ref, bond_ref, w3t_ref, sums_ref):
    gl, gh = _pre_activation(g2_ref, p1_ref, bond_ref, w3t_ref)

    @pl.when(pl.program_id(0) == 0)
    def _():
        sums_ref[...] = jnp.zeros((8, C), jnp.float32)

    sums_ref[0:1, :AD] += jnp.sum(gl, axis=0, keepdims=True)
    sums_ref[0:1, AD:] += jnp.sum(gh, axis=0, keepdims=True)
    sums_ref[1:2, :AD] += jnp.sum(gl * gl, axis=0, keepdims=True)
    sums_ref[1:2, AD:] += jnp.sum(gh * gh, axis=0, keepdims=True)


def _stats(g2h, p1, bond2, w3t, half):
    a0 = half * GRID_H
    return pl.pallas_call(
        _stats_body,
        grid=(GRID_H,),
        in_specs=[
            pl.BlockSpec((R_TILE, AD), lambda i: (i, 0)),
            pl.BlockSpec((A_TILE, C), lambda i: (i + a0, 0)),
            pl.BlockSpec((R_TILE, BD), lambda i: (i + a0, 0)),
            pl.BlockSpec((BD, C), lambda i: (0, 0)),
        ],
        out_specs=pl.BlockSpec((8, C), lambda i: (0, 0)),
        out_shape=jax.ShapeDtypeStruct((8, C), jnp.float32),
    )(g2h, p1, bond2, w3t)


def _apply_body(g2_ref, p1_ref, bond_ref, w3t_ref, sa_ref, sb_ref, s_ref, sums2_ref):
    gl, gh = _pre_activation(g2_ref, p1_ref, bond_ref, w3t_ref)
    sums = sa_ref[...] + sb_ref[...]
    mean = sums[0:1, :] * (1.0 / NM)
    var = sums[1:2, :] * (1.0 / NM) - mean * mean
    rstd = lax.rsqrt(var + EPS)
    rl = rstd[:, :AD]
    af = rl * (-LOG2E)
    cf = mean[:, :AD] * rl * LOG2E
    filt = 1.0 / (1.0 + jnp.exp2(gl * af + cf))
    rh = rstd[:, AD:]
    y = gh * rh - mean[:, AD:] * rh
    t = jnp.exp2(jnp.abs(y) * (-LOG2E))
    core = jnp.maximum(y, 0.0) + jnp.log(1.0 + t)
    prod = filt * core
    s = jnp.sum(prod.reshape(A_TILE, M, AD), axis=1)
    s_ref[...] = s

    @pl.when(pl.program_id(0) == 0)
    def _():
        sums2_ref[...] = jnp.zeros((8, AD), jnp.float32)

    sums2_ref[0:1, :] += jnp.sum(s, axis=0, keepdims=True)
    sums2_ref[1:2, :] += jnp.sum(s * s, axis=0, keepdims=True)


def _apply(g2h, p1, bond2, w3t, sums_a, sums_b, half):
    a0 = half * GRID_H
    return pl.pallas_call(
        _apply_body,
        grid=(GRID_H,),
        in_specs=[
            pl.BlockSpec((R_TILE, AD), lambda i: (i, 0)),
            pl.BlockSpec((A_TILE, C), lambda i: (i + a0, 0)),
            pl.BlockSpec((R_TILE, BD), lambda i: (i + a0, 0)),
            pl.BlockSpec((BD, C), lambda i: (0, 0)),
            pl.BlockSpec((8, C), lambda i: (0, 0)),
            pl.BlockSpec((8, C), lambda i: (0, 0)),
        ],
        out_specs=[
            pl.BlockSpec((A_TILE, AD), lambda i: (i, 0)),
            pl.BlockSpec((8, AD), lambda i: (0, 0)),
        ],
        out_shape=[
            jax.ShapeDtypeStruct((N_H, AD), jnp.float32),
            jax.ShapeDtypeStruct((8, AD), jnp.float32),
        ],
    )(g2h, p1, bond2, w3t, sums_a, sums_b)


def _final_body(atom_ref, s_ref, s2a_ref, s2b_ref, out_ref):
    sums2 = s2a_ref[...] + s2b_ref[...]
    mean = sums2[0:1, :] * (1.0 / N)
    var = sums2[1:2, :] * (1.0 / N) - mean * mean
    rstd = lax.rsqrt(var + EPS)
    sn = (s_ref[...] - mean) * rstd
    x = atom_ref[...] + sn
    out_ref[...] = jnp.maximum(x, 0.0) + jnp.log1p(jnp.exp2(jnp.abs(x) * (-LOG2E)))


def _final(atom, s_h, sums2_a, sums2_b, half):
    rows = 1000
    a0 = half * (N_H // rows)
    return pl.pallas_call(
        _final_body,
        grid=(N_H // rows,),
        in_specs=[
            pl.BlockSpec((rows, AD), lambda i: (i + a0, 0)),
            pl.BlockSpec((rows, AD), lambda i: (i, 0)),
            pl.BlockSpec((8, AD), lambda i: (0, 0)),
            pl.BlockSpec((8, AD), lambda i: (0, 0)),
        ],
        out_specs=pl.BlockSpec((rows, AD), lambda i: (i, 0)),
        out_shape=jax.ShapeDtypeStruct((N_H, AD), jnp.float32),
    )(atom, s_h, sums2_a, sums2_b)


def kernel(atom, bond, bond_idx, W, b):
    w1t = W[:, :AD].T.astype(jnp.float32)
    w2t = W[:, AD:2 * AD].T.astype(jnp.float32)
    w3t = W[:, 2 * AD:].T.astype(jnp.float32)
    b2 = b.reshape(1, C)

    p1, p2 = _project(atom, w1t, w2t, b2)
    bond2 = bond.reshape(NM, BD)
    idx2 = bond_idx.reshape(HALVES, NM_H)
    g2a = _sc_gather(idx2[0], p2)
    g2b = _sc_gather(idx2[1], p2)
    sums_a = _stats(g2a, p1, bond2, w3t, 0)
    sums_b = _stats(g2b, p1, bond2, w3t, 1)
    s_a, sums2_a = _apply(g2a, p1, bond2, w3t, sums_a, sums_b, 0)
    s_b, sums2_b = _apply(g2b, p1, bond2, w3t, sums_a, sums_b, 1)
    out_a = _final(atom, s_a, sums2_a, sums2_b, 0)
    out_b = _final(atom, s_b, sums2_a, sums2_b, 1)
    return jnp.concatenate([out_a, out_b], axis=0)


# speedup vs baseline: 3.3578x; 1.0598x over previous
import jax
import jax.numpy as jnp
from jax import lax
from jax.experimental import pallas as pl
from jax.experimental.pallas import tpu as pltpu
from jax.experimental.pallas import tpu_sc as plsc

N = 10000
M = 32
AD = 128
BD = 16
C = 2 * AD
NM = N * M
EPS = 1e-5
LOG2E = 1.4426950408889634

HALVES = 2
N_H = N // HALVES
NM_H = NM // HALVES

A_TILE = 200
R_TILE = A_TILE * M
GRID_H = N_H // A_TILE

SC_CORES = 2
SC_SUBCORES = 16
NW = SC_CORES * SC_SUBCORES
ROWS_PER_W = NM_H // NW
CHUNK = 40
CHUNKS = ROWS_PER_W // CHUNK


def _pack_bf16_pair(lo, hi):
    ul = lax.bitcast_convert_type(lo, jnp.uint32)
    uh = lax.bitcast_convert_type(hi, jnp.uint32)
    bl = (ul + jnp.uint32(0x7FFF) + ((ul >> 16) & jnp.uint32(1))) >> 16
    bh = (uh + jnp.uint32(0x7FFF) + ((uh >> 16) & jnp.uint32(1))) >> 16
    return lax.bitcast_convert_type(bl | (bh << 16), jnp.int32)


def _unpack_bf16_pair(w):
    u = lax.bitcast_convert_type(w, jnp.uint32)
    lo = lax.bitcast_convert_type(u << 16, jnp.float32)
    hi = lax.bitcast_convert_type(u, jnp.float32)
    return lo, hi


def _proj_body(atom_ref, w1t_ref, w2t_ref, b_ref, p1_ref, p2_ref):
    x = atom_ref[...]
    p1_ref[...] = (
        jnp.dot(x, w1t_ref[...], preferred_element_type=jnp.float32) + b_ref[...]
    )
    p2 = jnp.dot(x, w2t_ref[...], preferred_element_type=jnp.float32)
    p2_ref[...] = _pack_bf16_pair(p2[:, :AD], p2[:, AD:])


def _project(atom, w1t, w2t, b2):
    rows = 1000
    return pl.pallas_call(
        _proj_body,
        grid=(N // rows,),
        in_specs=[
            pl.BlockSpec((rows, AD), lambda i: (i, 0)),
            pl.BlockSpec((AD, C), lambda i: (0, 0)),
            pl.BlockSpec((AD, C), lambda i: (0, 0)),
            pl.BlockSpec((1, C), lambda i: (0, 0)),
        ],
        out_specs=[
            pl.BlockSpec((rows, C), lambda i: (i, 0)),
            pl.BlockSpec((rows, AD), lambda i: (i, 0)),
        ],
        out_shape=[
            jax.ShapeDtypeStruct((N, C), jnp.float32),
            jax.ShapeDtypeStruct((N, AD), jnp.int32),
        ],
    )(atom, w1t, w2t, b2)


def _gather_body(idx_hbm, p2_hbm, out_hbm, idx_v,
                 b0, b1, b2, b3, g0, g1, g2, g3, w0, w1, w2, w3):
    wid = lax.axis_index("s") * SC_CORES + lax.axis_index("c")
    base = wid * ROWS_PER_W
    bufs = (b0, b1, b2, b3)
    gsem = (g0, g1, g2, g3)
    wsem = (w0, w1, w2, w3)
    pltpu.sync_copy(idx_hbm.at[pl.ds(base, ROWS_PER_W)], idx_v)

    def gd(c, k):
        off = pl.multiple_of(c * CHUNK, 8)
        return pltpu.make_async_copy(
            p2_hbm.at[idx_v.at[pl.ds(off, CHUNK)]], bufs[k], gsem[k]
        )

    def wd(c, k):
        return pltpu.make_async_copy(
            bufs[k], out_hbm.at[pl.ds(base + c * CHUNK, CHUNK)], wsem[k]
        )

    def chunk(c, k, wait_prev, start_next):
        gd(c, k).wait()
        wd(c, k).start()
        if wait_prev:
            wd(c - 2, (k - 2) % 4).wait()
        if start_next:
            gd(c + 2, (k + 2) % 4).start()

    gd(0, 0).start()
    gd(1, 1).start()
    chunk(0, 0, False, True)
    chunk(1, 1, False, True)
    chunk(2, 2, True, True)
    chunk(3, 3, True, True)

    def step(t, carry):
        c0 = 4 + 4 * t
        chunk(c0, 0, True, True)
        chunk(c0 + 1, 1, True, True)
        chunk(c0 + 2, 2, True, True)
        chunk(c0 + 3, 3, True, True)
        return carry

    lax.fori_loop(0, (CHUNKS - 9) // 4, step, 0)
    c0 = CHUNKS - 5
    chunk(c0, 0, True, True)
    chunk(c0 + 1, 1, True, True)
    chunk(c0 + 2, 2, True, True)
    chunk(c0 + 3, 3, True, False)
    chunk(c0 + 4, 0, True, False)
    wd(c0 + 3, 3).wait()
    wd(c0 + 4, 0).wait()


def _sc_gather(flat_idx_half, p2):
    mesh = plsc.VectorSubcoreMesh(core_axis_name="c", subcore_axis_name="s")
    f = pl.kernel(
        _gather_body,
        out_type=jax.ShapeDtypeStruct((NM_H, AD), jnp.int32),
        mesh=mesh,
        scratch_types=(
            [pltpu.VMEM((ROWS_PER_W,), jnp.int32)]
            + [pltpu.VMEM((CHUNK, AD), jnp.int32) for _ in range(4)]
            + [pltpu.SemaphoreType.DMA for _ in range(8)]
        ),
    )
    return f(flat_idx_half, p2)


def _pre_activation(g2_ref, p1_ref, bond_ref, w3t_ref):
    p1 = p1_ref[...]
    p1rep = jnp.broadcast_to(p1[:, None, :], (A_TILE, M, C)).reshape(R_TILE, C)
    bw = jnp.dot(
        bond_ref[...].astype(jnp.float32),
        w3t_ref[...],
        preferred_element_type=jnp.float32,
    )
    base = p1rep + bw
    lo, hi = _unpack_bf16_pair(g2_ref[...])
    return lo + base[:, :AD], hi + base[:, AD:]


def _stats_body(g2_ref, p1_ref, bond_ref, w3t_ref, sums_ref):
    gl, gh = _pre_activation(g2_ref, p1_ref, bond_ref, w3t_ref)

    @pl.when(pl.program_id(0) == 0)
    def _():
        sums_ref[...] = jnp.zeros((8, C), jnp.float32)

    sums_ref[0:1, :AD] += jnp.sum(gl, axis=0, keepdims=True)
    sums_ref[0:1, AD:] += jnp.sum(gh, axis=0, keepdims=True)
    sums_ref[1:2, :AD] += jnp.sum(gl * gl, axis=0, keepdims=True)
    sums_ref[1:2, AD:] += jnp.sum(gh * gh, axis=0, keepdims=True)


def _stats(g2h, p1, bond2, w3t, half):
    a0 = half * GRID_H
    return pl.pallas_call(
        _stats_body,
        grid=(GRID_H,),
        in_specs=[
            pl.BlockSpec((R_TILE, AD), lambda i: (i, 0)),
            pl.BlockSpec((A_TILE, C), lambda i: (i + a0, 0)),
            pl.BlockSpec((R_TILE, BD), lambda i: (i + a0, 0)),
            pl.BlockSpec((BD, C), lambda i: (0, 0)),
        ],
        out_specs=pl.BlockSpec((8, C), lambda i: (0, 0)),
        out_shape=jax.ShapeDtypeStruct((8, C), jnp.float32),
    )(g2h, p1, bond2, w3t)


def _apply_body(g2_ref, p1_ref, bond_ref, w3t_ref, sa_ref, sb_ref, s_ref, sums2_ref):
    gl, gh = _pre_activation(g2_ref, p1_ref, bond_ref, w3t_ref)
    sums = sa_ref[...] + sb_ref[...]
    mean = sums[0:1, :] * (1.0 / NM)
    var = sums[1:2, :] * (1.0 / NM) - mean * mean
    rstd = lax.rsqrt(var + EPS)
    rl = rstd[:, :AD]
    af = rl * (-LOG2E)
    cf = mean[:, :AD] * rl * LOG2E
    filt = 1.0 / (1.0 + jnp.exp2(gl * af + cf))
    rh = rstd[:, AD:]
    y = gh * rh - mean[:, AD:] * rh
    t = jnp.exp2(jnp.abs(y) * (-LOG2E))
    core = jnp.maximum(y, 0.0) + jnp.log(1.0 + t)
    prod = filt * core
    s = jnp.sum(prod.reshape(A_TILE, M, AD), axis=1)
    s_ref[...] = s

    @pl.when(pl.program_id(0) == 0)
    def _():
        sums2_ref[...] = jnp.zeros((8, AD), jnp.float32)

    sums2_ref[0:1, :] += jnp.sum(s, axis=0, keepdims=True)
    sums2_ref[1:2, :] += jnp.sum(s * s, axis=0, keepdims=True)


def _apply(g2h, p1, bond2, w3t, sums_a, sums_b, half):
    a0 = half * GRID_H
    return pl.pallas_call(
        _apply_body,
        grid=(GRID_H,),
        in_specs=[
            pl.BlockSpec((R_TILE, AD), lambda i: (i, 0)),
            pl.BlockSpec((A_TILE, C), lambda i: (i + a0, 0)),
            pl.BlockSpec((R_TILE, BD), lambda i: (i + a0, 0)),
            pl.BlockSpec((BD, C), lambda i: (0, 0)),
            pl.BlockSpec((8, C), lambda i: (0, 0)),
            pl.BlockSpec((8, C), lambda i: (0, 0)),
        ],
        out_specs=[
            pl.BlockSpec((A_TILE, AD), lambda i: (i, 0)),
            pl.BlockSpec((8, AD), lambda i: (0, 0)),
        ],
        out_shape=[
            jax.ShapeDtypeStruct((N_H, AD), jnp.float32),
            jax.ShapeDtypeStruct((8, AD), jnp.float32),
        ],
    )(g2h, p1, bond2, w3t, sums_a, sums_b)


def _final_body(atom_ref, s_ref, s2a_ref, s2b_ref, out_ref):
    sums2 = s2a_ref[...] + s2b_ref[...]
    mean = sums2[0:1, :] * (1.0 / N)
    var = sums2[1:2, :] * (1.0 / N) - mean * mean
    rstd = lax.rsqrt(var + EPS)
    sn = (s_ref[...] - mean) * rstd
    x = atom_ref[...] + sn
    out_ref[...] = jnp.maximum(x, 0.0) + jnp.log1p(jnp.exp2(jnp.abs(x) * (-LOG2E)))


def _final(atom, s_h, sums2_a, sums2_b, half):
    rows = 1000
    a0 = half * (N_H // rows)
    return pl.pallas_call(
        _final_body,
        grid=(N_H // rows,),
        in_specs=[
            pl.BlockSpec((rows, AD), lambda i: (i + a0, 0)),
            pl.BlockSpec((rows, AD), lambda i: (i, 0)),
            pl.BlockSpec((8, AD), lambda i: (0, 0)),
            pl.BlockSpec((8, AD), lambda i: (0, 0)),
        ],
        out_specs=pl.BlockSpec((rows, AD), lambda i: (i, 0)),
        out_shape=jax.ShapeDtypeStruct((N_H, AD), jnp.float32),
    )(atom, s_h, sums2_a, sums2_b)


def kernel(atom, bond, bond_idx, W, b):
    w1t = W[:, :AD].T.astype(jnp.float32)
    w2t = W[:, AD:2 * AD].T.astype(jnp.float32)
    w3t = W[:, 2 * AD:].T.astype(jnp.float32)
    b2 = b.reshape(1, C)

    p1, p2 = _project(atom, w1t, w2t, b2)
    bond2 = bond.reshape(NM, BD).astype(jnp.bfloat16)
    idx2 = bond_idx.reshape(HALVES, NM_H)
    g2a = _sc_gather(idx2[0], p2)
    g2b = _sc_gather(idx2[1], p2)
    sums_a = _stats(g2a, p1, bond2, w3t, 0)
    sums_b = _stats(g2b, p1, bond2, w3t, 1)
    s_a, sums2_a = _apply(g2a, p1, bond2, w3t, sums_a, sums_b, 0)
    s_b, sums2_b = _apply(g2b, p1, bond2, w3t, sums_a, sums_b, 1)
    out_a = _final(atom, s_a, sums2_a, sums2_b, 0)
    out_b = _final(atom, s_b, sums2_a, sums2_b, 1)
    return jnp.concatenate([out_a, out_b], axis=0)
